# Initial kernel scaffold; baseline (speedup 1.0000x reference)
import jax
import jax.numpy as jnp
from jax.experimental import pallas as pl


def kernel(bilateral_grid, feature_map):
    raise NotImplementedError("write your pallas kernel here")


# speedup vs baseline: 210.8107x; 210.8107x over previous
import functools

import numpy as np
import jax
import jax.numpy as jnp
from jax import lax
from jax.experimental import pallas as pl
from jax.experimental.pallas import tpu as pltpu, tpu_sc as plsc

B, C, D, GH, GW = 4, 12, 8, 16, 16
H = W = 512
NC, NS, L = 2, 16, 16
NW = NC * NS
ROWS_PER_W = (B * H) // NW
PV = W // L

_wgf = (np.arange(W, dtype=np.float32) / np.float32(W - 1)) * np.float32(2.0) - np.float32(1.0)
_ix = np.clip((_wgf + np.float32(1.0)) * np.float32(0.5) * np.float32(GW - 1),
              0.0, np.float32(GW - 1)).astype(np.float32)
_x0 = np.minimum(np.floor(_ix).astype(np.int32), GW - 2)
_TX = _ix - _x0.astype(np.float32)
_XB = (_x0 * D).astype(np.int32)


def _body(gt_hbm, fm_hbm, tx_hbm, xb_hbm, out_hbm,
          gridv, fmv, txv, xbv, lv, obuf):
    wid = lax.axis_index("s") * NC + lax.axis_index("c")
    row0 = wid * ROWS_PER_W
    b = row0 // H
    h0 = row0 % H

    pltpu.sync_copy(tx_hbm, txv)
    pltpu.sync_copy(xb_hbm, xbv)
    pltpu.sync_copy(gt_hbm.at[b], gridv)
    pltpu.sync_copy(fm_hbm.at[pl.ds(row0, ROWS_PER_W)], fmv)

    def row_body(r, carry):
        h = h0 + r
        hgf = h.astype(jnp.float32) * np.float32(1.0 / (H - 1)) * np.float32(2.0) - np.float32(1.0)
        iy = jnp.clip((hgf + np.float32(1.0)) * np.float32(0.5) * np.float32(GH - 1),
                      np.float32(0.0), np.float32(GH - 1))
        iyv = jnp.full((L,), iy, dtype=jnp.float32)
        y0v = jnp.minimum(iyv.astype(jnp.int32), GH - 2)
        wy1 = iyv - y0v.astype(jnp.float32)
        wy0 = jnp.full((L,), 1.0, dtype=jnp.float32) - wy1
        y0 = jnp.max(y0v)
        ybase = y0 * (GW * D)

        for c in range(C):
            for xv in range(GW * D // L):
                off = ybase + (c * (GH * GW * D) + xv * L)
                g0 = gridv[pl.ds(off, L)]
                g1 = gridv[pl.ds(off + GW * D, L)]
                lv[pl.ds(c * (GW * D) + xv * L, L)] = wy0 * g0 + wy1 * g1

        def pv_body(p, carry2):
            s = p * L
            fme = fmv[r, pl.ds(s, L)]
            txe = txv[pl.ds(s, L)]
            xbe = xbv[pl.ds(s, L)]
            iz = jnp.clip((fme + np.float32(1.0)) * np.float32(0.5 * (D - 1)),
                          np.float32(0.0), np.float32(D - 1))
            z0 = jnp.minimum(iz.astype(jnp.int32), D - 2)
            tz = iz - z0.astype(jnp.float32)
            i00 = xbe + z0
            for c in range(C):
                idx = i00 + c * (GW * D)
                l0 = plsc.load_gather(lv, [idx])
                l1 = plsc.load_gather(lv, [idx + 1])
                r0 = plsc.load_gather(lv, [idx + D])
                r1 = plsc.load_gather(lv, [idx + (D + 1)])
                v0 = l0 + tz * (l1 - l0)
                v1 = r0 + tz * (r1 - r0)
                obuf[c, pl.ds(s, L)] = v0 + txe * (v1 - v0)
            return carry2

        lax.fori_loop(0, PV, pv_body, 0)
        pltpu.sync_copy(obuf, out_hbm.at[b, :, h, :])
        return carry

    lax.fori_loop(0, ROWS_PER_W, row_body, 0)


@jax.jit
def _slice_sc(gt, fm2, tx, xb):
    mesh = plsc.VectorSubcoreMesh(core_axis_name="c", subcore_axis_name="s")
    f = functools.partial(
        pl.kernel,
        out_type=jax.ShapeDtypeStruct((B, C, H, W), jnp.float32),
        mesh=mesh,
        compiler_params=pltpu.CompilerParams(needs_layout_passes=False),
        scratch_types=[
            pltpu.VMEM((C * GH * GW * D,), jnp.float32),
            pltpu.VMEM((ROWS_PER_W, W), jnp.float32),
            pltpu.VMEM((W,), jnp.float32),
            pltpu.VMEM((W,), jnp.int32),
            pltpu.VMEM((C * GW * D,), jnp.float32),
            pltpu.VMEM((C, W), jnp.float32),
        ],
    )(_body)
    return f(gt, fm2, tx, xb)


def kernel(bilateral_grid, feature_map):
    gt = jnp.transpose(bilateral_grid, (0, 1, 3, 4, 2)).reshape(B, C * GH * GW * D)
    fm2 = feature_map.reshape(B * H, W)
    return _slice_sc(gt, fm2, jnp.asarray(_TX), jnp.asarray(_XB))


# speedup vs baseline: 219.5184x; 1.0413x over previous
import functools

import numpy as np
import jax
import jax.numpy as jnp
from jax import lax
from jax.experimental import pallas as pl
from jax.experimental.pallas import tpu as pltpu, tpu_sc as plsc

B, C, D, GH, GW = 4, 12, 8, 16, 16
H = W = 512
NC, NS, L = 2, 16, 16
NW = NC * NS
ROWS_PER_W = (B * H) // NW
PV = W // L

_wgf = (np.arange(W, dtype=np.float32) / np.float32(W - 1)) * np.float32(2.0) - np.float32(1.0)
_ix = np.clip((_wgf + np.float32(1.0)) * np.float32(0.5) * np.float32(GW - 1),
              0.0, np.float32(GW - 1)).astype(np.float32)
_x0 = np.minimum(np.floor(_ix).astype(np.int32), GW - 2)
_TX = _ix - _x0.astype(np.float32)
_XB = (_x0 * D).astype(np.int32)


def _body(gt_hbm, fm_hbm, tx_hbm, xb_hbm, out_hbm,
          gridv, fmv, txv, xbv, lv, obuf, osem):
    wid = lax.axis_index("s") * NC + lax.axis_index("c")
    row0 = wid * ROWS_PER_W
    b = row0 // H
    h0 = row0 % H

    pltpu.sync_copy(tx_hbm, txv)
    pltpu.sync_copy(xb_hbm, xbv)
    pltpu.sync_copy(gt_hbm.at[b], gridv)
    pltpu.sync_copy(fm_hbm.at[pl.ds(row0, ROWS_PER_W)], fmv)

    def row_body(r, carry):
        h = h0 + r
        buf = lax.rem(r, 2)
        @pl.when(r >= 2)
        def _():
            pltpu.make_async_copy(obuf.at[buf], out_hbm.at[b, :, h, :], osem).wait()
        hgf = h.astype(jnp.float32) * np.float32(1.0 / (H - 1)) * np.float32(2.0) - np.float32(1.0)
        iy = jnp.clip((hgf + np.float32(1.0)) * np.float32(0.5) * np.float32(GH - 1),
                      np.float32(0.0), np.float32(GH - 1))
        iyv = jnp.full((L,), iy, dtype=jnp.float32)
        y0v = jnp.minimum(iyv.astype(jnp.int32), GH - 2)
        wy1 = iyv - y0v.astype(jnp.float32)
        wy0 = jnp.full((L,), 1.0, dtype=jnp.float32) - wy1
        y0 = jnp.max(y0v)
        ybase = y0 * (GW * D)

        for c in range(C):
            for xv in range(GW * D // L):
                off = ybase + (c * (GH * GW * D) + xv * L)
                g0 = gridv[pl.ds(off, L)]
                g1 = gridv[pl.ds(off + GW * D, L)]
                lv[pl.ds(c * (GW * D) + xv * L, L)] = wy0 * g0 + wy1 * g1

        def pv_body(p, carry2):
            s = p * L
            fme = fmv[r, pl.ds(s, L)]
            txe = txv[pl.ds(s, L)]
            xbe = xbv[pl.ds(s, L)]
            iz = jnp.clip((fme + np.float32(1.0)) * np.float32(0.5 * (D - 1)),
                          np.float32(0.0), np.float32(D - 1))
            z0 = jnp.minimum(iz.astype(jnp.int32), D - 2)
            tz = iz - z0.astype(jnp.float32)
            i00 = xbe + z0
            i01 = i00 + 1
            i10 = i00 + D
            i11 = i00 + (D + 1)
            for c in range(C):
                lc = lv.at[pl.ds(c * (GW * D), GW * D)]
                l0 = plsc.load_gather(lc, [i00])
                l1 = plsc.load_gather(lc, [i01])
                r0 = plsc.load_gather(lc, [i10])
                r1 = plsc.load_gather(lc, [i11])
                v0 = l0 + tz * (l1 - l0)
                v1 = r0 + tz * (r1 - r0)
                obuf[buf, c, pl.ds(s, L)] = v0 + txe * (v1 - v0)
            return carry2

        lax.fori_loop(0, PV, pv_body, 0)
        pltpu.make_async_copy(obuf.at[buf], out_hbm.at[b, :, h, :], osem).start()
        return carry

    lax.fori_loop(0, ROWS_PER_W, row_body, 0)
    for tail in (ROWS_PER_W - 2, ROWS_PER_W - 1):
        pltpu.make_async_copy(
            obuf.at[tail % 2], out_hbm.at[b, :, h0 + tail, :], osem).wait()


@jax.jit
def _slice_sc(gt, fm2, tx, xb):
    mesh = plsc.VectorSubcoreMesh(core_axis_name="c", subcore_axis_name="s")
    f = functools.partial(
        pl.kernel,
        out_type=jax.ShapeDtypeStruct((B, C, H, W), jnp.float32),
        mesh=mesh,
        compiler_params=pltpu.CompilerParams(needs_layout_passes=False),
        scratch_types=[
            pltpu.VMEM((C * GH * GW * D,), jnp.float32),
            pltpu.VMEM((ROWS_PER_W, W), jnp.float32),
            pltpu.VMEM((W,), jnp.float32),
            pltpu.VMEM((W,), jnp.int32),
            pltpu.VMEM((C * GW * D,), jnp.float32),
            pltpu.VMEM((2, C, W), jnp.float32),
            pltpu.SemaphoreType.DMA,
        ],
    )(_body)
    return f(gt, fm2, tx, xb)


def kernel(bilateral_grid, feature_map):
    gt = jnp.transpose(bilateral_grid, (0, 1, 3, 4, 2)).reshape(B, C * GH * GW * D)
    fm2 = feature_map.reshape(B * H, W)
    return _slice_sc(gt, fm2, jnp.asarray(_TX), jnp.asarray(_XB))


# speedup vs baseline: 399.8236x; 1.8214x over previous
import functools

import numpy as np
import jax
import jax.numpy as jnp
from jax import lax
from jax.experimental import pallas as pl
from jax.experimental.pallas import tpu as pltpu, tpu_sc as plsc

B, C, D, GH, GW = 4, 12, 8, 16, 16
H = W = 512
NC, NS, L = 2, 16, 16
NW = NC * NS
ROWS_PER_W = (B * H) // NW
PV = W // L

_wgf = (np.arange(W, dtype=np.float32) / np.float32(W - 1)) * np.float32(2.0) - np.float32(1.0)
_ix = np.clip((_wgf + np.float32(1.0)) * np.float32(0.5) * np.float32(GW - 1),
              0.0, np.float32(GW - 1)).astype(np.float32)
_x0 = np.minimum(np.floor(_ix).astype(np.int32), GW - 2)
_TX = _ix - _x0.astype(np.float32)
_XB = (_x0 * D).astype(np.int32)


def _body(gt_hbm, fm_hbm, tx_hbm, xb_hbm, out_hbm,
          gridv, fmv, txv, xbv, lv, obuf, osem):
    wid = lax.axis_index("s") * NC + lax.axis_index("c")
    row0 = wid * ROWS_PER_W
    b = row0 // H
    h0 = row0 % H

    pltpu.sync_copy(tx_hbm, txv)
    pltpu.sync_copy(xb_hbm, xbv)
    pltpu.sync_copy(gt_hbm.at[b], gridv)
    pltpu.sync_copy(fm_hbm.at[pl.ds(row0, ROWS_PER_W)], fmv)

    def row_body(r, carry):
        h = h0 + r
        buf = lax.rem(r, 2)
        @pl.when(r >= 2)
        def _():
            pltpu.make_async_copy(obuf.at[buf], out_hbm.at[b, :, h, :], osem).wait()
        hgf = h.astype(jnp.float32) * np.float32(1.0 / (H - 1)) * np.float32(2.0) - np.float32(1.0)
        iy = jnp.clip((hgf + np.float32(1.0)) * np.float32(0.5) * np.float32(GH - 1),
                      np.float32(0.0), np.float32(GH - 1))
        iyv = jnp.full((L,), iy, dtype=jnp.float32)
        y0v = jnp.minimum(iyv.astype(jnp.int32), GH - 2)
        wy1 = iyv - y0v.astype(jnp.float32)
        wy0 = jnp.full((L,), 1.0, dtype=jnp.float32) - wy1
        y0 = jnp.max(y0v)
        ybase = y0 * (GW * D)

        for c in range(C):
            for xv in range(GW * D // L):
                off = ybase + (c * (GH * GW * D) + xv * L)
                g0 = gridv[pl.ds(off, L)]
                g1 = gridv[pl.ds(off + GW * D, L)]
                lv[pl.ds(c * (GW * D) + xv * L, L)] = wy0 * g0 + wy1 * g1

        @plsc.parallel_loop(0, PV, unroll=4)
        def pv_body(p):
            s = p * L
            fme = fmv[r, pl.ds(s, L)]
            txe = txv[pl.ds(s, L)]
            xbe = xbv[pl.ds(s, L)]
            iz = jnp.clip((fme + np.float32(1.0)) * np.float32(0.5 * (D - 1)),
                          np.float32(0.0), np.float32(D - 1))
            z0 = jnp.minimum(iz.astype(jnp.int32), D - 2)
            tz = iz - z0.astype(jnp.float32)
            i00 = xbe + z0
            i01 = i00 + 1
            i10 = i00 + D
            i11 = i00 + (D + 1)
            one = jnp.full((L,), 1.0, dtype=jnp.float32)
            uz = one - tz
            ux = one - txe
            w00 = ux * uz
            w01 = ux * tz
            w10 = txe * uz
            w11 = txe * tz
            for c in range(C):
                lc = lv.at[pl.ds(c * (GW * D), GW * D)]
                l0 = plsc.load_gather(lc, [i00])
                l1 = plsc.load_gather(lc, [i01])
                r0 = plsc.load_gather(lc, [i10])
                r1 = plsc.load_gather(lc, [i11])
                obuf[buf, c, pl.ds(s, L)] = (
                    (w00 * l0 + w01 * l1) + (w10 * r0 + w11 * r1))
        pltpu.make_async_copy(obuf.at[buf], out_hbm.at[b, :, h, :], osem).start()
        return carry

    lax.fori_loop(0, ROWS_PER_W, row_body, 0)
    for tail in (ROWS_PER_W - 2, ROWS_PER_W - 1):
        pltpu.make_async_copy(
            obuf.at[tail % 2], out_hbm.at[b, :, h0 + tail, :], osem).wait()


@jax.jit
def _slice_sc(gt, fm2, tx, xb):
    mesh = plsc.VectorSubcoreMesh(core_axis_name="c", subcore_axis_name="s")
    f = functools.partial(
        pl.kernel,
        out_type=jax.ShapeDtypeStruct((B, C, H, W), jnp.float32),
        mesh=mesh,
        compiler_params=pltpu.CompilerParams(needs_layout_passes=False),
        scratch_types=[
            pltpu.VMEM((C * GH * GW * D,), jnp.float32),
            pltpu.VMEM((ROWS_PER_W, W), jnp.float32),
            pltpu.VMEM((W,), jnp.float32),
            pltpu.VMEM((W,), jnp.int32),
            pltpu.VMEM((C * GW * D,), jnp.float32),
            pltpu.VMEM((2, C, W), jnp.float32),
            pltpu.SemaphoreType.DMA,
        ],
    )(_body)
    return f(gt, fm2, tx, xb)


def kernel(bilateral_grid, feature_map):
    gt = jnp.transpose(bilateral_grid, (0, 1, 3, 4, 2)).reshape(B, C * GH * GW * D)
    fm2 = feature_map.reshape(B * H, W)
    return _slice_sc(gt, fm2, jnp.asarray(_TX), jnp.asarray(_XB))


# speedup vs baseline: 588.1367x; 1.4710x over previous
import functools

import numpy as np
import jax
import jax.numpy as jnp
from jax import lax
from jax.experimental import pallas as pl
from jax.experimental.pallas import tpu as pltpu, tpu_sc as plsc

B, C, D, GH, GW = 4, 12, 8, 16, 16
H = W = 512
NC, NS, L = 2, 16, 16
NW = NC * NS
ROWS_PER_W = (B * H) // NW
PV = W // L

_wgf = (np.arange(W, dtype=np.float32) / np.float32(W - 1)) * np.float32(2.0) - np.float32(1.0)
_ix = np.clip((_wgf + np.float32(1.0)) * np.float32(0.5) * np.float32(GW - 1),
              0.0, np.float32(GW - 1)).astype(np.float32)
_x0 = np.minimum(np.floor(_ix).astype(np.int32), GW - 2)
_TX = _ix - _x0.astype(np.float32)
_XB = (_x0 * D).astype(np.int32)


def _body(gt_hbm, fm_hbm, tx_hbm, xb_hbm, out_hbm,
          gridv, fmv, txv, xbv, lv, obuf, osem):
    wid = lax.axis_index("s") * NC + lax.axis_index("c")
    row0 = wid * ROWS_PER_W
    b = row0 // H
    h0 = row0 % H

    pltpu.sync_copy(tx_hbm, txv)
    pltpu.sync_copy(xb_hbm, xbv)
    pltpu.sync_copy(gt_hbm.at[b], gridv)
    pltpu.sync_copy(fm_hbm.at[pl.ds(row0, ROWS_PER_W)], fmv)

    def row_body(r, carry):
        h = h0 + r
        buf = lax.rem(r, 2)
        @pl.when(r >= 2)
        def _():
            pltpu.make_async_copy(obuf.at[buf], out_hbm.at[b, :, h, :], osem).wait()
        hgf = h.astype(jnp.float32) * np.float32(1.0 / (H - 1)) * np.float32(2.0) - np.float32(1.0)
        iy = jnp.clip((hgf + np.float32(1.0)) * np.float32(0.5) * np.float32(GH - 1),
                      np.float32(0.0), np.float32(GH - 1))
        iyv = jnp.full((L,), iy, dtype=jnp.float32)
        y0v = jnp.minimum(iyv.astype(jnp.int32), GH - 2)
        wy1 = iyv - y0v.astype(jnp.float32)
        wy0 = jnp.full((L,), 1.0, dtype=jnp.float32) - wy1
        y0 = jnp.max(y0v)
        ybase = y0 * (GW * D)

        for c in range(C):
            for xv in range(GW * D // L):
                off = ybase + (c * (GH * GW * D) + xv * L)
                g0 = gridv[pl.ds(off, L)]
                g1 = gridv[pl.ds(off + GW * D, L)]
                lv[pl.ds(c * (GW * D) + xv * L, L)] = wy0 * g0 + wy1 * g1

        @plsc.parallel_loop(0, PV, unroll=8)
        def pv_body(p):
            s = p * L
            fme = fmv[r, pl.ds(s, L)]
            txe = txv[pl.ds(s, L)]
            xbe = xbv[pl.ds(s, L)]
            iz = jnp.clip((fme + np.float32(1.0)) * np.float32(0.5 * (D - 1)),
                          np.float32(0.0), np.float32(D - 1))
            z0 = jnp.minimum(iz.astype(jnp.int32), D - 2)
            tz = iz - z0.astype(jnp.float32)
            i00 = xbe + z0
            i01 = i00 + 1
            i10 = i00 + D
            i11 = i00 + (D + 1)
            one = jnp.full((L,), 1.0, dtype=jnp.float32)
            uz = one - tz
            ux = one - txe
            w00 = ux * uz
            w01 = ux * tz
            w10 = txe * uz
            w11 = txe * tz
            for c in range(C):
                lc = lv.at[pl.ds(c * (GW * D), GW * D)]
                l0 = plsc.load_gather(lc, [i00])
                l1 = plsc.load_gather(lc, [i01])
                r0 = plsc.load_gather(lc, [i10])
                r1 = plsc.load_gather(lc, [i11])
                obuf[buf, c, pl.ds(s, L)] = (
                    (w00 * l0 + w01 * l1) + (w10 * r0 + w11 * r1))
        pltpu.make_async_copy(obuf.at[buf], out_hbm.at[b, :, h, :], osem).start()
        return carry

    lax.fori_loop(0, ROWS_PER_W, row_body, 0)
    for tail in (ROWS_PER_W - 2, ROWS_PER_W - 1):
        pltpu.make_async_copy(
            obuf.at[tail % 2], out_hbm.at[b, :, h0 + tail, :], osem).wait()


@jax.jit
def _slice_sc(gt, fm2, tx, xb):
    mesh = plsc.VectorSubcoreMesh(core_axis_name="c", subcore_axis_name="s")
    f = functools.partial(
        pl.kernel,
        out_type=jax.ShapeDtypeStruct((B, C, H, W), jnp.float32),
        mesh=mesh,
        compiler_params=pltpu.CompilerParams(needs_layout_passes=False),
        scratch_types=[
            pltpu.VMEM((C * GH * GW * D,), jnp.float32),
            pltpu.VMEM((ROWS_PER_W, W), jnp.float32),
            pltpu.VMEM((W,), jnp.float32),
            pltpu.VMEM((W,), jnp.int32),
            pltpu.VMEM((C * GW * D,), jnp.float32),
            pltpu.VMEM((2, C, W), jnp.float32),
            pltpu.SemaphoreType.DMA,
        ],
    )(_body)
    return f(gt, fm2, tx, xb)


def kernel(bilateral_grid, feature_map):
    gt = jnp.transpose(bilateral_grid, (0, 1, 3, 4, 2)).reshape(B, C * GH * GW * D)
    fm2 = feature_map.reshape(B * H, W)
    return _slice_sc(gt, fm2, jnp.asarray(_TX), jnp.asarray(_XB))
